# baseline (speedup 1.0000x reference)
import jax
import jax.numpy as jnp
from jax.experimental import pallas as pl
from jax.experimental.pallas import tpu as pltpu


def _dma_copy(in_ref, out_ref, sem):
    copy = pltpu.make_async_copy(in_ref, out_ref, sem)
    copy.start()
    copy.wait()


def kernel(table):
    n, d = table.shape
    return pl.pallas_call(
        _dma_copy,
        in_specs=[pl.BlockSpec(memory_space=pl.MemorySpace.ANY)],
        out_specs=pl.BlockSpec(memory_space=pl.MemorySpace.ANY),
        out_shape=jax.ShapeDtypeStruct((n, d), table.dtype),
        scratch_shapes=[pltpu.SemaphoreType.DMA],
    )(table)


# speedup vs baseline: 1.0020x; 1.0020x over previous
import jax
import jax.numpy as jnp
from jax.experimental import pallas as pl
from jax.experimental.pallas import tpu as pltpu

_STRIPS = 32


def _dma_copy(in_ref, out_ref, sem):
    n = in_ref.shape[0]
    rows = n // _STRIPS
    copies = []
    for i in range(_STRIPS):
        sl = pl.ds(i * rows, rows)
        copies.append(pltpu.make_async_copy(in_ref.at[sl], out_ref.at[sl], sem.at[i]))
    for c in copies:
        c.start()
    for c in copies:
        c.wait()


def kernel(table):
    n, d = table.shape
    return pl.pallas_call(
        _dma_copy,
        in_specs=[pl.BlockSpec(memory_space=pl.MemorySpace.ANY)],
        out_specs=pl.BlockSpec(memory_space=pl.MemorySpace.ANY),
        out_shape=jax.ShapeDtypeStruct((n, d), table.dtype),
        scratch_shapes=[pltpu.SemaphoreType.DMA((_STRIPS,))],
    )(table)


# speedup vs baseline: 44.3263x; 44.2401x over previous
import jax
import jax.numpy as jnp
from jax.experimental import pallas as pl

_BLOCK_ROWS = 1024


def _copy_block(in_ref, out_ref):
    out_ref[...] = in_ref[...]


def kernel(table):
    n, d = table.shape
    return pl.pallas_call(
        _copy_block,
        grid=(n // _BLOCK_ROWS,),
        in_specs=[pl.BlockSpec((_BLOCK_ROWS, d), lambda i: (i, 0))],
        out_specs=pl.BlockSpec((_BLOCK_ROWS, d), lambda i: (i, 0)),
        out_shape=jax.ShapeDtypeStruct((n, d), table.dtype),
    )(table)


# speedup vs baseline: 47.0961x; 1.0625x over previous
import jax
import jax.numpy as jnp
from jax.experimental import pallas as pl

_BLOCK_ROWS = 2048


def _copy_block(in_ref, out_ref):
    out_ref[...] = in_ref[...]


def kernel(table):
    n, d = table.shape
    return pl.pallas_call(
        _copy_block,
        grid=(n // _BLOCK_ROWS,),
        in_specs=[pl.BlockSpec((_BLOCK_ROWS, d), lambda i: (i, 0))],
        out_specs=pl.BlockSpec((_BLOCK_ROWS, d), lambda i: (i, 0)),
        out_shape=jax.ShapeDtypeStruct((n, d), table.dtype),
    )(table)
